# baseline (speedup 1.0000x reference)
import jax
import jax.numpy as jnp
from jax import lax
from jax.experimental import pallas as pl
from jax.experimental.pallas import tpu as pltpu

_NBITS = 8
_HIDDEN = 32
_PRED_CH = 1 + _NBITS
_ROWS_PER_STEP = 256


def _wam_kernel(imgs_ref, mask_ref, wimg_ref, wout_ref, bout_ref,
                wdet_ref, bdet_ref, imgs_w_ref, comb_ref, preds_ref):
    C = imgs_ref.shape[1]
    hb = imgs_ref.shape[2]
    W = imgs_ref.shape[3]
    P = hb * W

    wimg = wimg_ref[0]
    wout = wout_ref[...].astype(jnp.bfloat16)
    bout = bout_ref[...]
    wdet = wdet_ref[...].astype(jnp.bfloat16)
    bdet = bdet_ref[...]

    rc = 32
    Pc = rc * W
    ones = jnp.ones((1, Pc), jnp.float32)

    def body(i, carry):
        r0 = pl.multiple_of(i * rc, rc)
        x = imgs_ref[0, :, pl.ds(r0, rc), :].reshape(C, Pc)
        m = mask_ref[0, :, pl.ds(r0, rc), :].reshape(1, Pc)
        xa = jnp.concatenate([x, ones], axis=0)

        hpre = jnp.dot(wimg, xa, preferred_element_type=jnp.float32)
        h = jnp.tanh(hpre.astype(jnp.bfloat16))
        delta = jnp.tanh(
            jnp.dot(wout, h, preferred_element_type=jnp.float32) + bout)

        iw = (x + delta).astype(jnp.bfloat16)
        cb = (x + m * delta).astype(jnp.bfloat16)
        preds = (jnp.dot(wdet, cb, preferred_element_type=jnp.float32)
                 + bdet).astype(jnp.bfloat16)

        imgs_w_ref[0, :, pl.ds(r0, rc), :] = iw.reshape(C, rc, W)
        comb_ref[0, :, pl.ds(r0, rc), :] = cb.reshape(C, rc, W)
        preds_ref[0, :, pl.ds(r0, rc), :] = preds.reshape(_PRED_CH, rc, W)
        return carry

    lax.fori_loop(0, hb // rc, body, 0, unroll=2)


def _pick_rows(h):
    for hb in (_ROWS_PER_STEP, 128, 64, 32, 16, 8):
        if h % hb == 0:
            return hb
    return h


def kernel(imgs, masks, msgs, w_img_t, w_msg, b_h, w_out_t, b_out,
           w_det_t, b_det):
    B, C, H, W = imgs.shape
    hb = _pick_rows(H)
    n_rt = H // hb

    msg_pm1 = 2.0 * msgs.astype(jnp.float32) - 1.0
    mbias = (msg_pm1 @ w_msg + b_h).reshape(B, _HIDDEN, 1)
    wimg_aug = jnp.concatenate(
        [jnp.broadcast_to(w_img_t.astype(jnp.float32)[None],
                          (B, _HIDDEN, C)), mbias], axis=2)

    def band_map(i):
        return (i // n_rt, 0, i % n_rt, 0)

    def batch_map(i):
        return (i // n_rt, 0, 0)

    def weight_map(i):
        return (0, 0)

    in_specs = [
        pl.BlockSpec((1, C, hb, W), band_map),
        pl.BlockSpec((1, 1, hb, W), band_map),
        pl.BlockSpec((1, _HIDDEN, C + 1), batch_map),
        pl.BlockSpec(w_out_t.shape, weight_map),
        pl.BlockSpec(b_out.shape, weight_map),
        pl.BlockSpec(w_det_t.shape, weight_map),
        pl.BlockSpec(b_det.shape, weight_map),
    ]
    out_specs = (
        pl.BlockSpec((1, C, hb, W), band_map),
        pl.BlockSpec((1, C, hb, W), band_map),
        pl.BlockSpec((1, _PRED_CH, hb, W), band_map),
    )
    out_shapes = (
        jax.ShapeDtypeStruct((B, C, H, W), jnp.bfloat16),
        jax.ShapeDtypeStruct((B, C, H, W), jnp.bfloat16),
        jax.ShapeDtypeStruct((B, _PRED_CH, H, W), jnp.bfloat16),
    )

    return pl.pallas_call(
        _wam_kernel,
        out_shape=out_shapes,
        grid_spec=pltpu.PrefetchScalarGridSpec(
            num_scalar_prefetch=0,
            grid=(B * n_rt,),
            in_specs=in_specs,
            out_specs=out_specs),
        compiler_params=pltpu.CompilerParams(
            dimension_semantics=("parallel",)),
    )(imgs, masks, wimg_aug, w_out_t, b_out, w_det_t, b_det)


# speedup vs baseline: 1.1064x; 1.1064x over previous
import jax
import jax.numpy as jnp
from jax import lax
from jax.experimental import pallas as pl
from jax.experimental.pallas import tpu as pltpu

_NBITS = 8
_HIDDEN = 32
_PRED_CH = 1 + _NBITS
_ROWS_PER_STEP = 256


def _wam_kernel(imgs_ref, mask_ref, wimg_ref, wout_ref, bout_ref,
                wdet_ref, bdet_ref, imgs_w_ref, comb_ref, preds_ref):
    C = imgs_ref.shape[1]
    hb = imgs_ref.shape[2]
    W = imgs_ref.shape[3]
    P = hb * W

    wimg = wimg_ref[0]
    wout = wout_ref[...].astype(jnp.bfloat16)
    bout = bout_ref[...]
    wdet = wdet_ref[...].astype(jnp.bfloat16)
    bdet = bdet_ref[...]

    x = imgs_ref[0].reshape(C, P)
    m = mask_ref[0].reshape(1, P)
    xa = jnp.concatenate([x, jnp.ones((1, P), jnp.float32)], axis=0)

    hpre = jnp.dot(wimg, xa, preferred_element_type=jnp.float32)
    h = jnp.tanh(hpre.astype(jnp.bfloat16))
    delta = jnp.tanh(
        jnp.dot(wout, h, preferred_element_type=jnp.float32) + bout)

    iw = (x + delta).astype(jnp.bfloat16)
    cb = (x + m * delta).astype(jnp.bfloat16)
    preds = (jnp.dot(wdet, cb, preferred_element_type=jnp.float32)
             + bdet).astype(jnp.bfloat16)

    imgs_w_ref[0] = iw.reshape(C, hb, W)
    comb_ref[0] = cb.reshape(C, hb, W)
    preds_ref[0] = preds.reshape(_PRED_CH, hb, W)


def _pick_rows(h):
    for hb in (_ROWS_PER_STEP, 128, 64, 32, 16, 8):
        if h % hb == 0:
            return hb
    return h


def kernel(imgs, masks, msgs, w_img_t, w_msg, b_h, w_out_t, b_out,
           w_det_t, b_det):
    B, C, H, W = imgs.shape
    hb = _pick_rows(H)
    n_rt = H // hb

    msg_pm1 = 2.0 * msgs.astype(jnp.float32) - 1.0
    mbias = (msg_pm1 @ w_msg + b_h).reshape(B, _HIDDEN, 1)
    wimg_aug = (jnp.pad(w_img_t.astype(jnp.float32), ((0, 0), (0, 1)))[None]
                + jnp.pad(mbias, ((0, 0), (0, 0), (C, 0))))

    def band_map(i):
        return (i // n_rt, 0, i % n_rt, 0)

    def batch_map(i):
        return (i // n_rt, 0, 0)

    def weight_map(i):
        return (0, 0)

    in_specs = [
        pl.BlockSpec((1, C, hb, W), band_map),
        pl.BlockSpec((1, 1, hb, W), band_map),
        pl.BlockSpec((1, _HIDDEN, C + 1), batch_map),
        pl.BlockSpec(w_out_t.shape, weight_map),
        pl.BlockSpec(b_out.shape, weight_map),
        pl.BlockSpec(w_det_t.shape, weight_map),
        pl.BlockSpec(b_det.shape, weight_map),
    ]
    out_specs = (
        pl.BlockSpec((1, C, hb, W), band_map),
        pl.BlockSpec((1, C, hb, W), band_map),
        pl.BlockSpec((1, _PRED_CH, hb, W), band_map),
    )
    out_shapes = (
        jax.ShapeDtypeStruct((B, C, H, W), jnp.bfloat16),
        jax.ShapeDtypeStruct((B, C, H, W), jnp.bfloat16),
        jax.ShapeDtypeStruct((B, _PRED_CH, H, W), jnp.bfloat16),
    )

    return pl.pallas_call(
        _wam_kernel,
        out_shape=out_shapes,
        grid_spec=pltpu.PrefetchScalarGridSpec(
            num_scalar_prefetch=0,
            grid=(B * n_rt,),
            in_specs=in_specs,
            out_specs=out_specs),
        compiler_params=pltpu.CompilerParams(
            dimension_semantics=("parallel",)),
    )(imgs, masks, wimg_aug, w_out_t, b_out, w_det_t, b_det)
